# Initial kernel scaffold; baseline (speedup 1.0000x reference)
import jax
import jax.numpy as jnp
from jax.experimental import pallas as pl


def kernel(node_feats, edge_feats, params, edge_index, node_graph_ids):
    raise NotImplementedError("write your pallas kernel here")


# speedup vs baseline: 1.0842x; 1.0842x over previous
import functools
import jax
import jax.numpy as jnp
from jax import lax
from jax.experimental import pallas as pl
from jax.experimental.pallas import tpu as pltpu
from jax.experimental.pallas import tpu_sc as plsc

N = 10000
E = 320000
G = 256
DN = 128
DE = 16
H = 200
HP = 208
H3P = 3 * HP
NT = 8

BN = 400
BE = 2000

_f32 = jnp.float32


def _lrelu(x):
    return jnp.maximum(x, 0.01 * x)


def _elu(x):
    return jnp.where(x > 0, x, jnp.exp(jnp.minimum(x, 0.0)) - 1.0)


def _padw(w, r, c):
    return jnp.zeros((r, c), _f32).at[:w.shape[0], :w.shape[1]].set(w)


def _padv(b, n):
    return jnp.zeros((1, n), _f32).at[0, :b.shape[0]].set(b)


def _tile8(col):
    return jnp.tile(_padw(col[:, None], HP, 1), (1, 8))


def _pad_gru(Wi, Wh, bi, bh):
    WiP = jnp.zeros((HP, H3P), _f32)
    WhP = jnp.zeros((HP, H3P), _f32)
    biP = jnp.zeros((1, H3P), _f32)
    bhP = jnp.zeros((1, H3P), _f32)
    for k in range(3):
        WiP = WiP.at[:H, k * HP:k * HP + H].set(Wi[:, k * H:(k + 1) * H])
        WhP = WhP.at[:H, k * HP:k * HP + H].set(Wh[:, k * H:(k + 1) * H])
        biP = biP.at[0, k * HP:k * HP + H].set(bi[k * H:(k + 1) * H])
        bhP = bhP.at[0, k * HP:k * HP + H].set(bh[k * H:(k + 1) * H])
    return WiP, WhP, biP, bhP


def _gru_block(x, h, Wi, Wh, bi, bh):
    gi = jnp.dot(x, Wi, preferred_element_type=_f32) + bi
    gh = jnp.dot(h, Wh, preferred_element_type=_f32) + bh
    r = jax.nn.sigmoid(gi[:, :HP] + gh[:, :HP])
    z = jax.nn.sigmoid(gi[:, HP:2 * HP] + gh[:, HP:2 * HP])
    n = jnp.tanh(gi[:, 2 * HP:] + r * gh[:, 2 * HP:])
    return (1.0 - z) * n + z * h


def _full(shape):
    return pl.BlockSpec(shape, lambda *_: tuple(0 for _ in shape))


def _rows(bn, cols):
    return pl.BlockSpec((bn, cols), lambda i: (i, 0))


def _tc1_body(nf, pnW, pnb, pe1a, pe2a, pe2b, hv_o, nfp_o, q_o):
    x = nf[...]
    hv = _lrelu(jnp.dot(x, pnW[...], preferred_element_type=_f32) + pnb[...])
    hv_o[...] = hv
    nfp_o[...] = jnp.dot(x, pe1a[...], preferred_element_type=_f32)
    q_o[...] = jnp.dot(hv, pe2a[...], preferred_element_type=_f32) + pe2b[...]


def _tc2_body(ef, pe1b, pe1bb, efp_o):
    efp_o[...] = jnp.dot(ef[...], pe1b[...], preferred_element_type=_f32) + pe1bb[...]


def _tc3_body(S1a, S1b, saa, sab, hv, etW, etb, Wi, Wh, bi, bh, h_o):
    S1 = S1a[...] + S1b[...]
    sa = (saa[...] + sab[...])[:, :1]
    ctx = _elu(jnp.dot(S1, etW[...], preferred_element_type=_f32) + sa * etb[...])
    h_o[...] = jnp.maximum(
        _gru_block(ctx, hv[...], Wi[...], Wh[...], bi[...], bh[...]), 0.0)


def _tc4_body(h, uW, ub, vW, pnW, u_o, v_o, hq_o):
    x = h[...]
    u_o[...] = jnp.dot(x, uW[...], preferred_element_type=_f32) + ub[...]
    v_o[...] = jnp.dot(x, vW[...], preferred_element_type=_f32)
    hq_o[...] = jnp.dot(x, pnW[...], preferred_element_type=_f32)


def _tc5_body(S2a, S2b, saa, sab, h, pnb, Wi, Wh, bi, bh, h_o):
    sa = (saa[...] + sab[...])[:, :1]
    ctx2 = _elu(S2a[...] + S2b[...] + sa * pnb[...])
    h_o[...] = jnp.maximum(
        _gru_block(ctx2, h[...], Wi[...], Wh[...], bi[...], bh[...]), 0.0)


def _tc6_body(h2, gid, gf0_o):
    i = pl.program_id(0)

    @pl.when(i == 0)
    def _():
        gf0_o[...] = jnp.zeros_like(gf0_o)

    g = gid[0, 0, :]
    P = (lax.broadcasted_iota(jnp.int32, (G, BN), 0) == g[None, :]).astype(_f32)
    gf0_o[...] += jnp.dot(P, h2[...], preferred_element_type=_f32)


def _tc7_body(gf0, claW, tg_o):
    tg_o[...] = jnp.dot(jnp.maximum(gf0[...], 0.0), claW[...],
                        preferred_element_type=_f32)


def _tc8_body(h2, gid, tg, clWb, clb, roW, rob, npW, npb,
              ex3_o, hvr_o, nm_o, s3_o):
    i = pl.program_id(0)
    x = h2[...]
    g = gid[0, 0, :]
    Pt = (g[:, None] == lax.broadcasted_iota(jnp.int32, (BN, G), 1)).astype(_f32)
    tgn = jnp.dot(Pt, tg[...], preferred_element_type=_f32)
    th = jnp.dot(x, clWb[...], preferred_element_type=_f32) + clb[...]
    ex3 = jnp.exp(_lrelu(tgn + th))
    ex3_o[...] = ex3
    hvr_o[...] = jnp.dot(x, roW[...], preferred_element_type=_f32) + rob[...]
    nm_o[...] = jax.nn.sigmoid(jnp.dot(x, npW[...], preferred_element_type=_f32)
                               + npb[...])

    @pl.when(i == 0)
    def _():
        s3_o[...] = jnp.zeros_like(s3_o)

    P = (lax.broadcasted_iota(jnp.int32, (G, BN), 0) == g[None, :]).astype(_f32)
    s3_o[...] += jnp.dot(P, ex3, preferred_element_type=_f32)


def _tc9_body(ex3, hvr, gid, s3, grp_o):
    i = pl.program_id(0)
    g = gid[0, 0, :]
    s3inv = 1.0 / (s3[...] + 1e-9)
    Pt = (g[:, None] == lax.broadcasted_iota(jnp.int32, (BN, G), 1)).astype(_f32)
    s3n = jnp.dot(Pt, s3inv, preferred_element_type=_f32)
    an = (ex3[...] * s3n)[:, :1]

    @pl.when(i == 0)
    def _():
        grp_o[...] = jnp.zeros_like(grp_o)

    P = (lax.broadcasted_iota(jnp.int32, (G, BN), 0) == g[None, :]).astype(_f32)
    grp_o[...] += jnp.dot(P, an * hvr[...], preferred_element_type=_f32)


def _tc10_body(grp, gf0, Wi, Wh, bi, bh, predW, predb, pred_o, gfo_o):
    g_repr = _elu(grp[...])
    gfo = _gru_block(jnp.maximum(g_repr, 0.0), gf0[...],
                     Wi[...], Wh[...], bi[...], bh[...])
    gfo_o[...] = gfo
    pred_o[...] = jnp.dot(gfo, predW[...], preferred_element_type=_f32) + predb[...]


def _call(body, grid, in_specs, out_specs, out_shapes, *args):
    return pl.pallas_call(
        body, grid=grid, in_specs=in_specs, out_specs=out_specs,
        out_shape=out_shapes)(*args)


def kernel(node_feats, edge_feats, params, edge_index, node_graph_ids):
    p = params
    src = edge_index[0]
    dst = edge_index[1]
    gid = node_graph_ids
    gid3 = gid.reshape(N // BN, 1, BN)
    NB = N // BN

    pnW = _padw(p['gc_pn_W'], DN, HP)
    pnb = _padv(p['gc_pn_b'], HP)
    pe1a = _padw(p['gc_pe1_W'][:DN], DN, HP)
    pe1b = _padw(p['gc_pe1_W'][DN:], DE, HP)
    pe1bb = _padv(p['gc_pe1_b'], HP)
    pe2a = _tile8(p['gc_pe2_W'][:H, 0])
    pe2b = jnp.full((1, 8), p['gc_pe2_b'][0], _f32)
    w2 = _padv(p['gc_pe2_W'][H:, 0], HP)[0]
    etW = _padw(p['gc_et_W'], HP, HP)
    etb = _padv(p['gc_et_b'], HP)
    gru1 = _pad_gru(p['gc_gru_Wi'], p['gc_gru_Wh'], p['gc_gru_bi'], p['gc_gru_bh'])
    uW = _tile8(p['l1_pe_W'][:H, 0])
    ub = jnp.full((1, 8), p['l1_pe_b'][0], _f32)
    vW = _tile8(p['l1_pe_W'][H:, 0])
    l1pnW = _padw(p['l1_pn_W'], HP, HP)
    l1pnb = _padv(p['l1_pn_b'], HP)
    gru2 = _pad_gru(p['l1_gru_Wi'], p['l1_gru_Wh'], p['l1_gru_bi'], p['l1_gru_bh'])
    claW = _tile8(p['ro_cl_W'][:H, 0])
    clWb = _tile8(p['ro_cl_W'][H:, 0])
    clb = jnp.full((1, 8), p['ro_cl_b'][0], _f32)
    roW = _padw(p['ro_pn_W'], HP, HP)
    rob = _padv(p['ro_pn_b'], HP)
    gru3 = _pad_gru(p['ro_gru_Wi'], p['ro_gru_Wh'], p['ro_gru_bi'], p['ro_gru_bh'])
    predW = _padw(p['pred_W'], HP, 8)
    predb = _padv(p['pred_b'], 8)
    npW = _padw(p['np_W'], HP, 8)
    npb = jnp.full((1, 8), p['np_b'][0], _f32)

    fHP = _full((1, HP))
    f3 = _full((1, H3P))

    hv_new, nfp, q = _call(
        _tc1_body, (NB,),
        [_rows(BN, DN), _full((DN, HP)), fHP, _full((DN, HP)), _full((HP, 8)),
         _full((1, 8))],
        [_rows(BN, HP), _rows(BN, HP), _rows(BN, 8)],
        [jax.ShapeDtypeStruct((N, HP), _f32), jax.ShapeDtypeStruct((N, HP), _f32),
         jax.ShapeDtypeStruct((N, 8), _f32)],
        node_feats, pnW, pnb, pe1a, pe2a, pe2b)

    (efp,) = _call(
        _tc2_body, (E // BE,),
        [_rows(BE, DE), _full((DE, HP)), fHP],
        [_rows(BE, HP)],
        [jax.ShapeDtypeStruct((E, HP), _f32)],
        edge_feats, pe1b, pe1bb)

    qv = q[:, 0]
    m_e = _lrelu(nfp[src] + efp)
    r = m_e @ w2
    ex = jnp.exp(_lrelu(qv[dst] + r))
    s = jax.ops.segment_sum(ex, dst, num_segments=N)
    a = ex / (s + 1e-9)[dst]
    S1 = jax.ops.segment_sum(a[:, None] * m_e, dst, num_segments=N)
    sa = jax.ops.segment_sum(a, dst, num_segments=N)
    S1a, S1b = S1, jnp.zeros_like(S1)
    saa = jnp.tile(sa[:, None], (1, 8))
    sab = jnp.zeros_like(saa)

    (h,) = _call(
        _tc3_body, (NB,),
        [_rows(BN, HP), _rows(BN, HP), _rows(BN, 8), _rows(BN, 8), _rows(BN, HP),
         _full((HP, HP)), fHP, _full((HP, H3P)), _full((HP, H3P)), f3, f3],
        [_rows(BN, HP)],
        [jax.ShapeDtypeStruct((N, HP), _f32)],
        S1a, S1b, saa, sab, hv_new, etW, etb, *gru1)

    u, v, hq = _call(
        _tc4_body, (NB,),
        [_rows(BN, HP), _full((HP, 8)), _full((1, 8)), _full((HP, 8)),
         _full((HP, HP))],
        [_rows(BN, 8), _rows(BN, 8), _rows(BN, HP)],
        [jax.ShapeDtypeStruct((N, 8), _f32), jax.ShapeDtypeStruct((N, 8), _f32),
         jax.ShapeDtypeStruct((N, HP), _f32)],
        h, uW, ub, vW, l1pnW)

    ex2 = jnp.exp(_lrelu(u[:, 0][dst] + v[:, 0][src]))
    s2 = jax.ops.segment_sum(ex2, dst, num_segments=N)
    a2 = ex2 / (s2 + 1e-9)[dst]
    S2 = jax.ops.segment_sum(a2[:, None] * hq[src], dst, num_segments=N)
    sa2 = jax.ops.segment_sum(a2, dst, num_segments=N)
    S2a, S2b = S2, jnp.zeros_like(S2)
    sa2a = jnp.tile(sa2[:, None], (1, 8))
    sa2b = jnp.zeros_like(sa2a)

    (h2,) = _call(
        _tc5_body, (NB,),
        [_rows(BN, HP), _rows(BN, HP), _rows(BN, 8), _rows(BN, 8), _rows(BN, HP),
         fHP, _full((HP, H3P)), _full((HP, H3P)), f3, f3],
        [_rows(BN, HP)],
        [jax.ShapeDtypeStruct((N, HP), _f32)],
        S2a, S2b, sa2a, sa2b, h, l1pnb, *gru2)

    gid_spec = pl.BlockSpec((1, 1, BN), lambda i: (i, 0, 0))

    (gf0,) = _call(
        _tc6_body, (NB,),
        [_rows(BN, HP), gid_spec],
        [_full((G, HP))],
        [jax.ShapeDtypeStruct((G, HP), _f32)],
        h2, gid3)

    (tg,) = _call(
        _tc7_body, (1,),
        [_full((G, HP)), _full((HP, 8))],
        [_full((G, 8))],
        [jax.ShapeDtypeStruct((G, 8), _f32)],
        gf0, claW)

    ex3, hvr, nm, s3 = _call(
        _tc8_body, (NB,),
        [_rows(BN, HP), gid_spec, _full((G, 8)), _full((HP, 8)), _full((1, 8)),
         _full((HP, HP)), fHP, _full((HP, 8)), _full((1, 8))],
        [_rows(BN, 8), _rows(BN, HP), _rows(BN, 8), _full((G, 8))],
        [jax.ShapeDtypeStruct((N, 8), _f32), jax.ShapeDtypeStruct((N, HP), _f32),
         jax.ShapeDtypeStruct((N, 8), _f32), jax.ShapeDtypeStruct((G, 8), _f32)],
        h2, gid3, tg, clWb, clb, roW, rob, npW, npb)

    (grp,) = _call(
        _tc9_body, (NB,),
        [_rows(BN, 8), _rows(BN, HP), gid_spec, _full((G, 8))],
        [_full((G, HP))],
        [jax.ShapeDtypeStruct((G, HP), _f32)],
        ex3, hvr, gid3, s3)

    pred, gfo = _call(
        _tc10_body, (1,),
        [_full((G, HP)), _full((G, HP)), _full((HP, H3P)), _full((HP, H3P)),
         f3, f3, _full((HP, 8)), _full((1, 8))],
        [_full((G, 8)), _full((G, HP))],
        [jax.ShapeDtypeStruct((G, 8), _f32), jax.ShapeDtypeStruct((G, HP), _f32)],
        grp, gf0, *gru3, predW, predb)

    return pred[:, :NT], nm[:, :1], gfo[:, :H]


# speedup vs baseline: 7.0617x; 6.5132x over previous
import dataclasses
import functools
import jax
import jax.numpy as jnp
from jax import lax
from jax.experimental import pallas as pl
from jax.experimental.pallas import tpu as pltpu
from jax.experimental.pallas import tpu_sc as plsc

N = 10000
E = 320000
G = 256
DN = 128
DE = 16
H = 200
HP = 208
H3P = 3 * HP
NT = 8

BN = 400
BE = 2000

NC = 2
NS = 16
NW = NC * NS
L = 16
EW = E // NW
CC = 80
NCH = EW // CC
NGRP = CC // L
NV = HP // L
HPG = 256
SP = 10240
SPT = SP // NS

_f32 = jnp.float32
_i32 = jnp.int32


def _sc_mesh():
    return plsc.VectorSubcoreMesh(core_axis_name="c", subcore_axis_name="s",
                                  num_cores=NC, num_subcores=NS)


def _sc_cp():
    cp = pltpu.CompilerParams()
    if "needs_layout_passes" in pltpu.CompilerParams.__dataclass_fields__:
        cp = dataclasses.replace(cp, needs_layout_passes=False)
    return cp


def _lrelu(x):
    return jnp.maximum(x, 0.01 * x)


def _elu(x):
    return jnp.where(x > 0, x, jnp.exp(jnp.minimum(x, 0.0)) - 1.0)


def _padw(w, r, c):
    return jnp.zeros((r, c), _f32).at[:w.shape[0], :w.shape[1]].set(w)


def _padv(b, n):
    return jnp.zeros((1, n), _f32).at[0, :b.shape[0]].set(b)


def _tile8(col):
    return jnp.tile(_padw(col[:, None], HP, 1), (1, 8))


def _pad_gru(Wi, Wh, bi, bh):
    WiP = jnp.zeros((HP, H3P), _f32)
    WhP = jnp.zeros((HP, H3P), _f32)
    biP = jnp.zeros((1, H3P), _f32)
    bhP = jnp.zeros((1, H3P), _f32)
    for k in range(3):
        WiP = WiP.at[:H, k * HP:k * HP + H].set(Wi[:, k * H:(k + 1) * H])
        WhP = WhP.at[:H, k * HP:k * HP + H].set(Wh[:, k * H:(k + 1) * H])
        biP = biP.at[0, k * HP:k * HP + H].set(bi[k * H:(k + 1) * H])
        bhP = bhP.at[0, k * HP:k * HP + H].set(bh[k * H:(k + 1) * H])
    return WiP, WhP, biP, bhP


def _gru_block(x, h, Wi, Wh, bi, bh):
    gi = jnp.dot(x, Wi, preferred_element_type=_f32) + bi
    gh = jnp.dot(h, Wh, preferred_element_type=_f32) + bh
    r = jax.nn.sigmoid(gi[:, :HP] + gh[:, :HP])
    z = jax.nn.sigmoid(gi[:, HP:2 * HP] + gh[:, HP:2 * HP])
    n = jnp.tanh(gi[:, 2 * HP:] + r * gh[:, 2 * HP:])
    return (1.0 - z) * n + z * h


def _full(shape):
    return pl.BlockSpec(shape, lambda *_: tuple(0 for _ in shape))


def _rows(bn, cols):
    return pl.BlockSpec((bn, cols), lambda i: (i, 0))


def _tc1_body(nf, pnW, pnb, pe1a, pe2a, pe2b, hv_o, nfp_o, q_o):
    x = nf[...]
    hv = _lrelu(jnp.dot(x, pnW[...], preferred_element_type=_f32) + pnb[...])
    hv_o[...] = hv
    nfp_o[...] = jnp.dot(x, pe1a[...], preferred_element_type=_f32)
    q_o[...] = jnp.dot(hv, pe2a[...], preferred_element_type=_f32) + pe2b[...]


def _tc2_body(ef, pe1b, pe1bb, efp_o):
    efp_o[...] = jnp.dot(ef[...], pe1b[...], preferred_element_type=_f32) + pe1bb[...]


def _tc3_body(SA0, SA1, SB0, SB1, sp0, sp1, hv, etW, etb, Wi, Wh, bi, bh, h_o):
    st = sp0[0, 0, :] + sp1[0, 0, :]
    sv = (1.0 / (st + 1e-9))[:, None]
    U = jnp.concatenate([SA0[...] + SA1[...], SB0[...] + SB1[...]], axis=1)
    S1 = sv * U
    sa = st[:, None] * sv
    ctx = _elu(jnp.dot(S1, etW[...], preferred_element_type=_f32) + sa * etb[...])
    h_o[...] = jnp.maximum(
        _gru_block(ctx, hv[...], Wi[...], Wh[...], bi[...], bh[...]), 0.0)


def _tc4_body(h, uW, ub, vW, pnW, u_o, v_o, hqA_o, hqB_o):
    x = h[...]
    u_o[...] = jnp.dot(x, uW[...], preferred_element_type=_f32) + ub[...]
    v_o[...] = jnp.dot(x, vW[...], preferred_element_type=_f32)
    hq = jnp.dot(x, pnW[...], preferred_element_type=_f32)
    hqA_o[...] = hq[:, :128]
    hqB_o[...] = hq[:, 128:]


def _tc5_body(SA0, SA1, SB0, SB1, sp0, sp1, h, pnb, Wi, Wh, bi, bh, h_o):
    st = sp0[0, 0, :] + sp1[0, 0, :]
    sv = (1.0 / (st + 1e-9))[:, None]
    U = jnp.concatenate([SA0[...] + SA1[...], SB0[...] + SB1[...]], axis=1)
    sa = st[:, None] * sv
    ctx2 = _elu(sv * U[:, :HP] + sa * pnb[...])
    h_o[...] = jnp.maximum(
        _gru_block(ctx2, h[...], Wi[...], Wh[...], bi[...], bh[...]), 0.0)


def _tc6_body(h2, gid, gf0_o):
    i = pl.program_id(0)

    @pl.when(i == 0)
    def _():
        gf0_o[...] = jnp.zeros_like(gf0_o)

    g = gid[0, 0, :]
    P = (lax.broadcasted_iota(jnp.int32, (G, BN), 0) == g[None, :]).astype(_f32)
    gf0_o[...] += jnp.dot(P, h2[...], preferred_element_type=_f32)


def _tc7_body(gf0, claW, tg_o):
    tg_o[...] = jnp.dot(jnp.maximum(gf0[...], 0.0), claW[...],
                        preferred_element_type=_f32)


def _tc8_body(h2, gid, tg, clWb, clb, roW, rob, npW, npb,
              ex3_o, hvr_o, nm_o, s3_o):
    i = pl.program_id(0)
    x = h2[...]
    g = gid[0, 0, :]
    Pt = (g[:, None] == lax.broadcasted_iota(jnp.int32, (BN, G), 1)).astype(_f32)
    tgn = jnp.dot(Pt, tg[...], preferred_element_type=_f32)
    th = jnp.dot(x, clWb[...], preferred_element_type=_f32) + clb[...]
    ex3 = jnp.exp(_lrelu(tgn + th))
    ex3_o[...] = ex3
    hvr_o[...] = jnp.dot(x, roW[...], preferred_element_type=_f32) + rob[...]
    nm_o[...] = jax.nn.sigmoid(jnp.dot(x, npW[...], preferred_element_type=_f32)
                               + npb[...])

    @pl.when(i == 0)
    def _():
        s3_o[...] = jnp.zeros_like(s3_o)

    P = (lax.broadcasted_iota(jnp.int32, (G, BN), 0) == g[None, :]).astype(_f32)
    s3_o[...] += jnp.dot(P, ex3, preferred_element_type=_f32)


def _tc9_body(ex3, hvr, gid, s3, grp_o):
    i = pl.program_id(0)
    g = gid[0, 0, :]
    s3inv = 1.0 / (s3[...] + 1e-9)
    Pt = (g[:, None] == lax.broadcasted_iota(jnp.int32, (BN, G), 1)).astype(_f32)
    s3n = jnp.dot(Pt, s3inv, preferred_element_type=_f32)
    an = (ex3[...] * s3n)[:, :1]

    @pl.when(i == 0)
    def _():
        grp_o[...] = jnp.zeros_like(grp_o)

    P = (lax.broadcasted_iota(jnp.int32, (G, BN), 0) == g[None, :]).astype(_f32)
    grp_o[...] += jnp.dot(P, an * hvr[...], preferred_element_type=_f32)


def _tc10_body(grp, gf0, Wi, Wh, bi, bh, predW, predb, pred_o, gfo_o):
    g_repr = _elu(grp[...])
    gfo = _gru_block(jnp.maximum(g_repr, 0.0), gf0[...],
                     Wi[...], Wh[...], bi[...], bh[...])
    gfo_o[...] = gfo
    pred_o[...] = jnp.dot(gfo, predW[...], preferred_element_type=_f32) + predb[...]


def _call(body, grid, in_specs, out_specs, out_shapes, *args):
    return pl.pallas_call(
        body, grid=grid, in_specs=in_specs, out_specs=out_specs,
        out_shape=out_shapes)(*args)


def _zero_vec(ref, n):
    @pl.loop(0, n // L)
    def _(i):
        ref[pl.ds(i * L, L)] = jnp.zeros((L,), _f32)


def _zero_rows128(ref, rows):
    @pl.loop(0, rows)
    def _(r):
        for j in range(8):
            ref[r, pl.ds(j * L, L)] = jnp.zeros((L,), _f32)


def _wid_core_sid():
    core = lax.axis_index("c")
    sid = lax.axis_index("s")
    return sid * NC + core, core, sid


def _splat_i32(x):
    return jnp.full((L,), x, _i32)


def _sc_pass1(nfp, efp, qv, w2v, src3, dst3):
    @functools.partial(
        pl.kernel,
        out_type=[jax.ShapeDtypeStruct((E, HPG), _f32),
                  jax.ShapeDtypeStruct((E,), _f32),
                  jax.ShapeDtypeStruct((NC * SP,), _f32)],
        mesh=_sc_mesh(),
        compiler_params=_sc_cp(),
        scratch_types=[
            pltpu.VMEM((N,), _f32),
            pltpu.VMEM((EW,), _i32),
            pltpu.VMEM((NCH, CC), _i32),
            pltpu.VMEM((CC, HPG), _f32),
            pltpu.VMEM((CC, HP), _f32),
            pltpu.VMEM((CC, HPG), _f32),
            pltpu.VMEM((CC,), _f32),
            pltpu.VMEM((HP,), _f32),
            pltpu.VMEM((1024,), _f32),
            pltpu.VMEM_SHARED((SP,), _f32),
            pltpu.SemaphoreType.DMA,
            pltpu.SemaphoreType.DMA,
        ],
    )
    def k(nfp_h, efp_h, q_h, w2_h, src_h, dst_h,
          m_h, ex_h, sp_h,
          q_v, src_v, dst_v, rows_v, erows_v, m_v, ex_v, w2_v, z_v, s_sh,
          sem, sem2):
        wid, core, sid = _wid_core_sid()
        pltpu.sync_copy(q_h, q_v)
        pltpu.sync_copy(src_h.at[wid], src_v)
        pltpu.sync_copy(dst_h.at[wid], dst_v)
        pltpu.sync_copy(w2_h, w2_v)
        _zero_vec(z_v, 1024)
        pltpu.sync_copy(z_v.at[pl.ds(0, SPT)], s_sh.at[pl.ds(sid * SPT, SPT)])

        @pl.loop(0, CC)
        def _(r):
            for j in range(NV, HPG // L):
                m_v[r, pl.ds(j * L, L)] = jnp.zeros((L,), _f32)

        plsc.subcore_barrier()

        w2r = [w2_v[pl.ds(j * L, L)] for j in range(NV)]
        liota = jnp.arange(L, dtype=_i32)

        @pl.loop(0, NCH)
        def _chunk(c):
            base = wid * EW + c * CC
            pltpu.async_copy(nfp_h.at[src_v.at[pl.ds(c * CC, CC)]],
                             rows_v, sem).wait()
            pltpu.async_copy(efp_h.at[pl.ds(base, CC), :], erows_v, sem2).wait()

            @pl.loop(0, NGRP)
            def _grp(g):
                rvec = jnp.zeros((L,), _f32)
                for lane in range(L):
                    er = g * L + lane
                    acc = jnp.zeros((L,), _f32)
                    for j in range(NV):
                        x = rows_v[er, pl.ds(j * L, L)] + erows_v[er, pl.ds(j * L, L)]
                        m = jnp.maximum(x, 0.01 * x)
                        m_v[er, pl.ds(j * L, L)] = m
                        acc = acc + m * w2r[j]
                    rvec = jnp.where(liota == lane, jnp.sum(acc), rvec)
                dg = dst_v[c, pl.ds(g * L, L)]
                qd = plsc.load_gather(q_v, [dg])
                y = qd + rvec
                ex_v[pl.ds(g * L, L)] = jnp.exp(jnp.maximum(y, 0.01 * y))
                for lane in range(L):
                    er = g * L + lane
                    es = plsc.load_gather(ex_v, [_splat_i32(er)])
                    for j in range(NV):
                        m_v[er, pl.ds(j * L, L)] = m_v[er, pl.ds(j * L, L)] * es

            pltpu.async_copy(m_v, m_h.at[pl.ds(base, CC), :], sem).wait()
            pltpu.async_copy(ex_v, ex_h.at[pl.ds(base, CC)], sem2).wait()
            pltpu.sync_copy(ex_v, s_sh.at[dst_v.at[c]], add=True)

        plsc.subcore_barrier()
        pltpu.sync_copy(s_sh.at[pl.ds(sid * SPT, SPT)], z_v.at[pl.ds(0, SPT)])
        pltpu.sync_copy(z_v.at[pl.ds(0, SPT)],
                        sp_h.at[pl.ds(core * SP + sid * SPT, SPT)])

    return k(nfp, efp, qv, w2v, src3, dst3)


def _sc_pass2(M, dst1):
    @functools.partial(
        pl.kernel,
        out_type=[jax.ShapeDtypeStruct((2, NC, N, 128), _f32)],
        mesh=_sc_mesh(),
        compiler_params=_sc_cp(),
        scratch_types=[
            pltpu.VMEM((CC, 128), _f32),
            pltpu.VMEM((CC, 128), _f32),
            pltpu.VMEM((CC,), _i32),
            pltpu.VMEM((CC,), _i32),
            pltpu.VMEM((48, 128), _f32),
            pltpu.VMEM_SHARED((N, 128), _f32),
            pltpu.SemaphoreType.DMA,
            pltpu.SemaphoreType.DMA,
            pltpu.SemaphoreType.DMA,
            pltpu.SemaphoreType.DMA,
        ],
    )
    def k(m_h, dst_h, s1_h,
          rows0, rows1, dst0, dst1v, z_v, SS_sh, smr0, smr1, smd0, smd1):
        wid, core, sid = _wid_core_sid()
        rows_b = (rows0, rows1)
        dst_b = (dst0, dst1v)
        smr = (smr0, smr1)
        smd = (smd0, smd1)

        for half in range(2):
            _zero_rows128(z_v, 48)

            @pl.loop(0, 13)
            def _(kk):
                pltpu.sync_copy(z_v, SS_sh.at[pl.ds(sid * 624 + kk * 48, 48), :])

            @pl.when(sid == 0)
            def _():
                pltpu.sync_copy(z_v.at[pl.ds(0, 16), :],
                                SS_sh.at[pl.ds(9984, 16), :])

            plsc.subcore_barrier()

            def start(c, b):
                base = wid * EW + c * CC
                pltpu.async_copy(
                    m_h.at[pl.ds(base, CC), pl.ds(half * 128, 128)],
                    rows_b[b], smr[b])
                pltpu.async_copy(dst_h.at[pl.ds(base, CC)], dst_b[b], smd[b])

            def wait(b):
                pltpu.make_async_copy(
                    m_h.at[pl.ds(0, CC), pl.ds(half * 128, 128)],
                    rows_b[b], smr[b]).wait()
                pltpu.make_async_copy(
                    dst_h.at[pl.ds(0, CC)], dst_b[b], smd[b]).wait()

            def scatter(b):
                pltpu.sync_copy(rows_b[b], SS_sh.at[dst_b[b]], add=True)

            start(0, 0)

            @pl.loop(0, NCH // 2)
            def _(c2):
                c0 = c2 * 2
                wait(0)
                start(c0 + 1, 1)
                scatter(0)
                wait(1)

                @pl.when(c2 + 1 < NCH // 2)
                def _():
                    start(c0 + 2, 0)

                scatter(1)

            start(NCH - 1, 0)
            wait(0)
            scatter(0)

            plsc.subcore_barrier()

            @pl.loop(0, 13)
            def _(kk):
                r0 = sid * 624 + kk * 48
                pltpu.sync_copy(SS_sh.at[pl.ds(r0, 48), :], z_v)
                pltpu.sync_copy(z_v, s1_h.at[half, core, pl.ds(r0, 48), :])

            @pl.when(sid == 0)
            def _():
                pltpu.sync_copy(SS_sh.at[pl.ds(9984, 16), :], z_v.at[pl.ds(0, 16), :])
                pltpu.sync_copy(z_v.at[pl.ds(0, 16), :],
                                s1_h.at[half, core, pl.ds(9984, 16), :])

            plsc.subcore_barrier()

    return k(M, dst1)[0]


def _sc_pass3(ut, vt, src3g, dst3):
    @functools.partial(
        pl.kernel,
        out_type=[jax.ShapeDtypeStruct((E,), _f32),
                  jax.ShapeDtypeStruct((NC * SP,), _f32)],
        mesh=_sc_mesh(),
        compiler_params=_sc_cp(),
        scratch_types=[
            pltpu.VMEM((N,), _f32),
            pltpu.VMEM((N,), _f32),
            pltpu.VMEM((NCH, CC), _i32),
            pltpu.VMEM((NCH, CC), _i32),
            pltpu.VMEM((CC,), _f32),
            pltpu.VMEM((1024,), _f32),
            pltpu.VMEM_SHARED((SP,), _f32),
            pltpu.SemaphoreType.DMA,
        ],
    )
    def k(u_h, v_h, src_h, dst_h,
          ex_h, sp_h,
          u_v, v_v, src_v, dst_v, ex_v, z_v, s_sh, sem):
        wid, core, sid = _wid_core_sid()
        pltpu.sync_copy(u_h, u_v)
        pltpu.sync_copy(v_h, v_v)
        pltpu.sync_copy(src_h.at[wid], src_v)
        pltpu.sync_copy(dst_h.at[wid], dst_v)
        _zero_vec(z_v, 1024)
        pltpu.sync_copy(z_v.at[pl.ds(0, SPT)], s_sh.at[pl.ds(sid * SPT, SPT)])

        plsc.subcore_barrier()

        @pl.loop(0, NCH)
        def _chunk(c):
            base = wid * EW + c * CC

            @pl.loop(0, NGRP)
            def _grp(g):
                sl = pl.ds(g * L, L)
                sg = src_v[c, sl]
                dg = dst_v[c, sl]
                us = plsc.load_gather(u_v, [dg])
                vs = plsc.load_gather(v_v, [sg])
                y = us + vs
                ex_v[sl] = jnp.exp(jnp.maximum(y, 0.01 * y))

            pltpu.async_copy(ex_v, ex_h.at[pl.ds(base, CC)], sem).wait()
            pltpu.sync_copy(ex_v, s_sh.at[dst_v.at[c]], add=True)

        plsc.subcore_barrier()
        pltpu.sync_copy(s_sh.at[pl.ds(sid * SPT, SPT)], z_v.at[pl.ds(0, SPT)])
        pltpu.sync_copy(z_v.at[pl.ds(0, SPT)],
                        sp_h.at[pl.ds(core * SP + sid * SPT, SPT)])

    return k(ut, vt, src3g, dst3)


def _sc_pass4(hqA, hqB, EX2, src1, dst1):
    @functools.partial(
        pl.kernel,
        out_type=[jax.ShapeDtypeStruct((2, NC, N, 128), _f32)],
        mesh=_sc_mesh(),
        compiler_params=_sc_cp(),
        scratch_types=[
            pltpu.VMEM((CC, 128), _f32),
            pltpu.VMEM((CC, 128), _f32),
            pltpu.VMEM((CC,), _i32),
            pltpu.VMEM((CC,), _i32),
            pltpu.VMEM((CC,), _i32),
            pltpu.VMEM((CC,), _i32),
            pltpu.VMEM((CC,), _f32),
            pltpu.VMEM((CC,), _f32),
            pltpu.VMEM((48, 128), _f32),
            pltpu.VMEM_SHARED((N, 128), _f32),
            pltpu.SemaphoreType.DMA,
            pltpu.SemaphoreType.DMA,
            pltpu.SemaphoreType.DMA,
            pltpu.SemaphoreType.DMA,
            pltpu.SemaphoreType.DMA,
            pltpu.SemaphoreType.DMA,
        ],
    )
    def k(hqA_h, hqB_h, ex_h, src_h, dst_h, s2_h,
          rows0, rows1, src0, src1v, dst0, dst1v, exb0, exb1, z_v, SS_sh,
          smr0, smr1, sms0, sms1, smd0, smd1):
        wid, core, sid = _wid_core_sid()
        rows_b = (rows0, rows1)
        src_b = (src0, src1v)
        dst_b = (dst0, dst1v)
        ex_b = (exb0, exb1)
        smr = (smr0, smr1)
        sms = (sms0, sms1)
        smd = (smd0, smd1)

        for half in range(2):
            tab = hqA_h if half == 0 else hqB_h
            _zero_rows128(z_v, 48)

            @pl.loop(0, 13)
            def _(kk):
                pltpu.sync_copy(z_v, SS_sh.at[pl.ds(sid * 624 + kk * 48, 48), :])

            @pl.when(sid == 0)
            def _():
                pltpu.sync_copy(z_v.at[pl.ds(0, 16), :],
                                SS_sh.at[pl.ds(9984, 16), :])

            plsc.subcore_barrier()

            def start_idx(c, b):
                base = wid * EW + c * CC
                pltpu.async_copy(src_h.at[pl.ds(base, CC)], src_b[b], sms[b])
                pltpu.async_copy(ex_h.at[pl.ds(base, CC)], ex_b[b], smd[b])
                pltpu.async_copy(dst_h.at[pl.ds(base, CC)], dst_b[b], smd[b])

            def wait_idx(b):
                pltpu.make_async_copy(src_h.at[pl.ds(0, CC)], src_b[b], sms[b]).wait()
                pltpu.make_async_copy(ex_h.at[pl.ds(0, CC)], ex_b[b], smd[b]).wait()
                pltpu.make_async_copy(dst_h.at[pl.ds(0, CC)], dst_b[b], smd[b]).wait()

            def gather(b):
                pltpu.async_copy(tab.at[src_b[b]], rows_b[b], smr[b])

            def wait_gather(b):
                pltpu.make_async_copy(tab.at[src_b[b]], rows_b[b], smr[b]).wait()

            def work(b):
                wait_gather(b)

                @pl.loop(0, CC)
                def _(e):
                    es = plsc.load_gather(ex_b[b], [_splat_i32(e)])
                    for j in range(8):
                        rows_b[b][e, pl.ds(j * L, L)] = (
                            rows_b[b][e, pl.ds(j * L, L)] * es)

                pltpu.sync_copy(rows_b[b], SS_sh.at[dst_b[b]], add=True)

            start_idx(0, 0)
            wait_idx(0)
            gather(0)

            @pl.loop(0, NCH // 2)
            def _(c2):
                c0 = c2 * 2
                start_idx(c0 + 1, 1)
                wait_idx(1)
                gather(1)
                work(0)

                @pl.when(c2 + 1 < NCH // 2)
                def _():
                    start_idx(c0 + 2, 0)
                    wait_idx(0)
                    gather(0)

                work(1)

            start_idx(NCH - 1, 0)
            wait_idx(0)
            gather(0)
            work(0)

            plsc.subcore_barrier()

            @pl.loop(0, 13)
            def _(kk):
                r0 = sid * 624 + kk * 48
                pltpu.sync_copy(SS_sh.at[pl.ds(r0, 48), :], z_v)
                pltpu.sync_copy(z_v, s2_h.at[half, core, pl.ds(r0, 48), :])

            @pl.when(sid == 0)
            def _():
                pltpu.sync_copy(SS_sh.at[pl.ds(9984, 16), :], z_v.at[pl.ds(0, 16), :])
                pltpu.sync_copy(z_v.at[pl.ds(0, 16), :],
                                s2_h.at[half, core, pl.ds(9984, 16), :])

            plsc.subcore_barrier()

    return k(hqA, hqB, EX2, src1, dst1)[0]


def kernel(node_feats, edge_feats, params, edge_index, node_graph_ids):
    p = params
    src = edge_index[0]
    dst = edge_index[1]
    gid = node_graph_ids
    gid3 = gid.reshape(N // BN, 1, BN)
    NB = N // BN

    pnW = _padw(p['gc_pn_W'], DN, HP)
    pnb = _padv(p['gc_pn_b'], HP)
    pe1a = _padw(p['gc_pe1_W'][:DN], DN, HPG)
    pe1b = _padw(p['gc_pe1_W'][DN:], DE, HP)
    pe1bb = _padv(p['gc_pe1_b'], HP)
    pe2a = _tile8(p['gc_pe2_W'][:H, 0])
    pe2b = jnp.full((1, 8), p['gc_pe2_b'][0], _f32)
    w2 = _padv(p['gc_pe2_W'][H:, 0], HP)[0]
    etW = _padw(p['gc_et_W'], HPG, HP)
    etb = _padv(p['gc_et_b'], HP)
    gru1 = _pad_gru(p['gc_gru_Wi'], p['gc_gru_Wh'], p['gc_gru_bi'], p['gc_gru_bh'])
    uW = _tile8(p['l1_pe_W'][:H, 0])
    ub = jnp.full((1, 8), p['l1_pe_b'][0], _f32)
    vW = _tile8(p['l1_pe_W'][H:, 0])
    l1pnW = _padw(p['l1_pn_W'], HP, HPG)
    l1pnb = _padv(p['l1_pn_b'], HP)
    gru2 = _pad_gru(p['l1_gru_Wi'], p['l1_gru_Wh'], p['l1_gru_bi'], p['l1_gru_bh'])
    claW = _tile8(p['ro_cl_W'][:H, 0])
    clWb = _tile8(p['ro_cl_W'][H:, 0])
    clb = jnp.full((1, 8), p['ro_cl_b'][0], _f32)
    roW = _padw(p['ro_pn_W'], HP, HP)
    rob = _padv(p['ro_pn_b'], HP)
    gru3 = _pad_gru(p['ro_gru_Wi'], p['ro_gru_Wh'], p['ro_gru_bi'], p['ro_gru_bh'])
    predW = _padw(p['pred_W'], HP, 8)
    predb = _padv(p['pred_b'], 8)
    npW = _padw(p['np_W'], HP, 8)
    npb = jnp.full((1, 8), p['np_b'][0], _f32)

    fHP = _full((1, HP))
    f3 = _full((1, H3P))

    hv_new, nfp, q = _call(
        _tc1_body, (NB,),
        [_rows(BN, DN), _full((DN, HP)), fHP, _full((DN, HPG)), _full((HP, 8)),
         _full((1, 8))],
        [_rows(BN, HP), _rows(BN, HPG), _rows(BN, 8)],
        [jax.ShapeDtypeStruct((N, HP), _f32), jax.ShapeDtypeStruct((N, HPG), _f32),
         jax.ShapeDtypeStruct((N, 8), _f32)],
        node_feats, pnW, pnb, pe1a, pe2a, pe2b)

    (efp,) = _call(
        _tc2_body, (E // BE,),
        [_rows(BE, DE), _full((DE, HP)), fHP],
        [_rows(BE, HP)],
        [jax.ShapeDtypeStruct((E, HP), _f32)],
        edge_feats, pe1b, pe1bb)

    src3 = src.reshape(NW, EW)
    dst3 = dst.reshape(NW, NCH, CC)
    M, EX, spart = _sc_pass1(nfp, efp, q[:, 0], w2, src3, dst3)
    S1part = _sc_pass2(M, dst)
    SA0, SA1 = S1part[0, 0], S1part[0, 1]
    SB0, SB1 = S1part[1, 0], S1part[1, 1]
    sp0 = spart[:N].reshape(NB, 1, BN)
    sp1 = spart[SP:SP + N].reshape(NB, 1, BN)

    sa_spec = pl.BlockSpec((1, 1, BN), lambda i: (i, 0, 0))

    (h,) = _call(
        _tc3_body, (NB,),
        [_rows(BN, 128), _rows(BN, 128), _rows(BN, 128), _rows(BN, 128),
         sa_spec, sa_spec, _rows(BN, HP),
         _full((HPG, HP)), fHP, _full((HP, H3P)), _full((HP, H3P)), f3, f3],
        [_rows(BN, HP)],
        [jax.ShapeDtypeStruct((N, HP), _f32)],
        SA0, SA1, SB0, SB1, sp0, sp1, hv_new, etW, etb, *gru1)

    u, v, hqA, hqB = _call(
        _tc4_body, (NB,),
        [_rows(BN, HP), _full((HP, 8)), _full((1, 8)), _full((HP, 8)),
         _full((HP, HPG))],
        [_rows(BN, 8), _rows(BN, 8), _rows(BN, 128), _rows(BN, 128)],
        [jax.ShapeDtypeStruct((N, 8), _f32), jax.ShapeDtypeStruct((N, 8), _f32),
         jax.ShapeDtypeStruct((N, 128), _f32),
         jax.ShapeDtypeStruct((N, 128), _f32)],
        h, uW, ub, vW, l1pnW)

    src3c = src.reshape(NW, NCH, CC)
    EX2, s2part = _sc_pass3(u[:, 0], v[:, 0], src3c, dst3)
    S2part = _sc_pass4(hqA, hqB, EX2, src, dst)
    TA0, TA1 = S2part[0, 0], S2part[0, 1]
    TB0, TB1 = S2part[1, 0], S2part[1, 1]
    tp0 = s2part[:N].reshape(NB, 1, BN)
    tp1 = s2part[SP:SP + N].reshape(NB, 1, BN)

    (h2,) = _call(
        _tc5_body, (NB,),
        [_rows(BN, 128), _rows(BN, 128), _rows(BN, 128), _rows(BN, 128),
         sa_spec, sa_spec, _rows(BN, HP),
         fHP, _full((HP, H3P)), _full((HP, H3P)), f3, f3],
        [_rows(BN, HP)],
        [jax.ShapeDtypeStruct((N, HP), _f32)],
        TA0, TA1, TB0, TB1, tp0, tp1, h, l1pnb, *gru2)

    gid_spec = pl.BlockSpec((1, 1, BN), lambda i: (i, 0, 0))

    (gf0,) = _call(
        _tc6_body, (NB,),
        [_rows(BN, HP), gid_spec],
        [_full((G, HP))],
        [jax.ShapeDtypeStruct((G, HP), _f32)],
        h2, gid3)

    (tg,) = _call(
        _tc7_body, (1,),
        [_full((G, HP)), _full((HP, 8))],
        [_full((G, 8))],
        [jax.ShapeDtypeStruct((G, 8), _f32)],
        gf0, claW)

    ex3, hvr, nm, s3 = _call(
        _tc8_body, (NB,),
        [_rows(BN, HP), gid_spec, _full((G, 8)), _full((HP, 8)), _full((1, 8)),
         _full((HP, HP)), fHP, _full((HP, 8)), _full((1, 8))],
        [_rows(BN, 8), _rows(BN, HP), _rows(BN, 8), _full((G, 8))],
        [jax.ShapeDtypeStruct((N, 8), _f32), jax.ShapeDtypeStruct((N, HP), _f32),
         jax.ShapeDtypeStruct((N, 8), _f32), jax.ShapeDtypeStruct((G, 8), _f32)],
        h2, gid3, tg, clWb, clb, roW, rob, npW, npb)

    (grp,) = _call(
        _tc9_body, (NB,),
        [_rows(BN, 8), _rows(BN, HP), gid_spec, _full((G, 8))],
        [_full((G, HP))],
        [jax.ShapeDtypeStruct((G, HP), _f32)],
        ex3, hvr, gid3, s3)

    pred, gfo = _call(
        _tc10_body, (1,),
        [_full((G, HP)), _full((G, HP)), _full((HP, H3P)), _full((HP, H3P)),
         f3, f3, _full((HP, 8)), _full((1, 8))],
        [_full((G, 8)), _full((G, HP))],
        [jax.ShapeDtypeStruct((G, 8), _f32), jax.ShapeDtypeStruct((G, HP), _f32)],
        grp, gf0, *gru3, predW, predb)

    return pred[:, :NT], nm[:, :1], gfo[:, :H]
